# Initial kernel scaffold; baseline (speedup 1.0000x reference)
import jax
import jax.numpy as jnp
from jax.experimental import pallas as pl


def kernel(inputs, w):
    raise NotImplementedError("write your pallas kernel here")


# speedup vs baseline: 1.4751x; 1.4751x over previous
import jax
import jax.numpy as jnp
from jax import lax
from jax.experimental import pallas as pl
from jax.experimental.pallas import tpu as pltpu
from jax.experimental.pallas import tpu_sc as plsc

BATCH = 16384
N_FIELDS = 26
NUM_CORES = 2
NUM_SUBCORES = 16
NUM_WORKERS = NUM_CORES * NUM_SUBCORES
ROWS_PER_W = BATCH // NUM_WORKERS
IDX_PER_W = ROWS_PER_W * N_FIELDS
LANES = 16


def _sc_body(w_hbm, idx_hbm, out_hbm, idx_v, rows_v, out_v, sem):
    wid = lax.axis_index("s") * NUM_CORES + lax.axis_index("c")
    base_i = wid * IDX_PER_W
    base_o = wid * ROWS_PER_W

    pltpu.sync_copy(idx_hbm.at[pl.ds(base_i, IDX_PER_W)], idx_v)
    pltpu.async_copy(w_hbm.at[idx_v], rows_v, sem).wait()

    @pl.loop(0, ROWS_PER_W // LANES)
    def _chunk(i):
        b = i * LANES
        acc = rows_v[pl.ds(b, LANES)]
        for f in range(1, N_FIELDS):
            acc = acc + rows_v[pl.ds(f * ROWS_PER_W + b, LANES)]
        out_v[pl.ds(b, LANES)] = acc

    pltpu.sync_copy(out_v, out_hbm.at[pl.ds(base_o, ROWS_PER_W)])


@jax.jit
def kernel(inputs, w):
    idx_flat = (
        inputs.astype(jnp.int32)
        .reshape(NUM_WORKERS, ROWS_PER_W, N_FIELDS)
        .transpose(0, 2, 1)
        .reshape(-1)
    )
    w_flat = w.reshape(-1)
    mesh = plsc.VectorSubcoreMesh(core_axis_name="c", subcore_axis_name="s")
    out = pl.kernel(
        _sc_body,
        out_type=jax.ShapeDtypeStruct((BATCH,), jnp.float32),
        mesh=mesh,
        scratch_types=[
            pltpu.VMEM((IDX_PER_W,), jnp.int32),
            pltpu.VMEM((IDX_PER_W,), jnp.float32),
            pltpu.VMEM((ROWS_PER_W,), jnp.float32),
            pltpu.SemaphoreType.DMA,
        ],
    )(w_flat, idx_flat)
    return out.reshape(BATCH, 1)


# speedup vs baseline: 1.4772x; 1.0015x over previous
import jax
import jax.numpy as jnp
from jax import lax
from jax.experimental import pallas as pl
from jax.experimental.pallas import tpu as pltpu
from jax.experimental.pallas import tpu_sc as plsc

FEATURE = 1000000
BATCH = 16384
N_FIELDS = 26
NUM_CORES = 2
NUM_SUBCORES = 16
NUM_WORKERS = NUM_CORES * NUM_SUBCORES
ROWS_PER_W = BATCH // NUM_WORKERS
IDX_PER_W = ROWS_PER_W * N_FIELDS
LANES = 16


def _sc_body(w_hbm, idx_hbm, out_hbm, idx_v, rows_bits, out_v, sem):
    wid = lax.axis_index("s") * NUM_CORES + lax.axis_index("c")
    base_i = wid * IDX_PER_W
    base_o = wid * ROWS_PER_W

    pltpu.sync_copy(idx_hbm.at[pl.ds(base_i, IDX_PER_W)], idx_v)
    pltpu.async_copy(w_hbm.at[idx_v], rows_bits, sem).wait()

    @pl.loop(0, ROWS_PER_W // LANES)
    def _chunk(i):
        b = i * LANES
        acc = plsc.bitcast(rows_bits[pl.ds(b, LANES)], jnp.float32)
        for f in range(1, N_FIELDS):
            acc = acc + plsc.bitcast(
                rows_bits[pl.ds(f * ROWS_PER_W + b, LANES)], jnp.float32
            )
        out_v[pl.ds(b, LANES)] = acc

    pltpu.sync_copy(out_v, out_hbm.at[pl.ds(base_o, ROWS_PER_W)])


@jax.jit
def kernel(inputs, w):
    idx_flat = (
        inputs.astype(jnp.int32)
        .reshape(NUM_WORKERS, ROWS_PER_W, N_FIELDS)
        .transpose(0, 2, 1)
        .reshape(-1)
    )
    w_bits_flat = lax.bitcast_convert_type(w, jnp.int32).reshape((FEATURE,))
    mesh = plsc.VectorSubcoreMesh(core_axis_name="c", subcore_axis_name="s")
    out = pl.kernel(
        _sc_body,
        out_type=jax.ShapeDtypeStruct((BATCH,), jnp.float32),
        mesh=mesh,
        scratch_types=[
            pltpu.VMEM((IDX_PER_W,), jnp.int32),
            pltpu.VMEM((IDX_PER_W,), jnp.int32),
            pltpu.VMEM((ROWS_PER_W,), jnp.float32),
            pltpu.SemaphoreType.DMA,
        ],
        compiler_params=pltpu.CompilerParams(needs_layout_passes=False),
    )(w_bits_flat, idx_flat)
    return out.reshape(BATCH, 1)
